# baseline (speedup 1.0000x reference)
import functools

import jax
import jax.numpy as jnp
from jax.experimental import pallas as pl
from jax.experimental.pallas import tpu as pltpu

EPS = 1e-5
CB = 32


def _fused_body(x_ref, m_ref, gb_ref, o_ref, acc_ref, *, cb, h, w, n):
    ph = pl.program_id(0)
    b = pl.program_id(1)
    i = pl.program_id(2)

    @pl.when(ph == 0)
    def _stats():
        x3 = x_ref[0, 0].reshape(cb, h, w)
        mb = m_ref[0, 0][None]
        x2 = x3 * x3
        xm = x3 * mb
        x2m = x2 * mb
        s_fg = jnp.sum(xm, axis=(1, 2), keepdims=True)
        q_fg = jnp.sum(x2m, axis=(1, 2), keepdims=True)
        s_all = jnp.sum(x3, axis=(1, 2), keepdims=True)
        q_all = jnp.sum(x2, axis=(1, 2), keepdims=True)
        cnt = jnp.sum(mb)
        part = jnp.concatenate(
            [s_fg, q_fg, s_all, q_all,
             jnp.full((cb, 1, 1), cnt, x3.dtype),
             jnp.zeros((cb, 1, 3), x3.dtype)], axis=2)

        @pl.when(jnp.logical_and(b == 0, i == 0))
        def _():
            acc_ref[...] = jnp.zeros_like(acc_ref)

        acc_ref[pl.ds(i * cb, cb)] += part

    @pl.when(ph == 1)
    def _apply():
        st = acc_ref[pl.ds(i * cb, cb)]
        s_fg = st[:, :, 0:1]
        q_fg = st[:, :, 1:2]
        s_all = st[:, :, 2:3]
        q_all = st[:, :, 3:4]
        cnt = st[:, :, 4:5]

        sr_fg = jnp.where(cnt == 0.0, 1.0, cnt)
        cnt_bg = n - cnt
        sr_bg = jnp.where(cnt_bg == 0.0, 1.0, cnt_bg)
        s_bg = s_all - s_fg
        q_bg = q_all - q_fg
        mu_fg = s_fg / sr_fg
        mu_bg = s_bg / sr_bg
        var_fg = (q_fg - sr_fg * mu_fg * mu_fg) / n
        var_bg = (q_bg - sr_bg * mu_bg * mu_bg) / n
        a_fg = jax.lax.rsqrt(var_fg + EPS) * jnp.sqrt(sr_fg / n)
        a_bg = jax.lax.rsqrt(var_bg + EPS) * jnp.sqrt(sr_bg / n)

        gb = gb_ref[0]
        A_fg = a_fg * (1.0 + gb[:, :, 0:1])
        A_bg = a_bg * (1.0 + gb[:, :, 2:3])
        beta = gb[:, :, 1:2] + gb[:, :, 3:4]
        D_fg = beta - mu_fg * A_fg
        D_bg = beta - mu_bg * A_bg

        x3 = x_ref[0, 0].reshape(cb, h, w)
        fg = m_ref[0, 0][None] != 0.0
        a_sel = jnp.where(fg, A_fg, A_bg)
        d_sel = jnp.where(fg, D_fg, D_bg)
        res = x3 * a_sel + d_sel
        o_ref[0, 0] = res.reshape(cb * h, w)


def kernel(x, mask, fg_gamma, fg_beta, bg_gamma, bg_beta):
    B, C, H, W = x.shape
    n = float(B * H * W)
    nc = C // CB
    x4 = x.reshape(B, nc, CB * H, W)
    gb = jnp.stack([fg_gamma, fg_beta, bg_gamma, bg_beta],
                   axis=1).reshape(nc, CB, 1, 4)

    out = pl.pallas_call(
        functools.partial(_fused_body, cb=CB, h=H, w=W, n=n),
        grid=(2, B, nc),
        in_specs=[
            pl.BlockSpec((1, 1, CB * H, W), lambda p, b, i: (b, i, 0, 0)),
            pl.BlockSpec((1, 1, H, W), lambda p, b, i: (b, 0, 0, 0)),
            pl.BlockSpec((1, CB, 1, 4), lambda p, b, i: (i, 0, 0, 0)),
        ],
        out_specs=pl.BlockSpec((1, 1, CB * H, W),
                               lambda p, b, i: (p * b, p * i, 0, 0)),
        out_shape=jax.ShapeDtypeStruct((B, nc, CB * H, W), jnp.float32),
        scratch_shapes=[pltpu.VMEM((C, 1, 8), jnp.float32)],
        compiler_params=pltpu.CompilerParams(
            dimension_semantics=("arbitrary", "arbitrary", "arbitrary"),
            vmem_limit_bytes=50 * 1024 * 1024,
        ),
    )(x4, mask, gb)

    return out.reshape(B, C, H, W)


# speedup vs baseline: 1.0039x; 1.0039x over previous
import functools

import jax
import jax.numpy as jnp
from jax.experimental import pallas as pl
from jax.experimental.pallas import tpu as pltpu

EPS = 1e-5
CB = 32


def _stats_body(x_ref, m_ref, o_ref, *, cb, h, w):
    b = pl.program_id(0)
    i = pl.program_id(1)
    x3 = x_ref[0, 0].reshape(cb, h, w)
    mb = m_ref[0, 0][None]
    x2 = x3 * x3
    xm = x3 * mb
    x2m = x2 * mb
    s_fg = jnp.sum(xm, axis=(1, 2), keepdims=True)
    q_fg = jnp.sum(x2m, axis=(1, 2), keepdims=True)
    s_all = jnp.sum(x3, axis=(1, 2), keepdims=True)
    q_all = jnp.sum(x2, axis=(1, 2), keepdims=True)
    cnt = jnp.sum(mb)
    part = jnp.concatenate(
        [s_fg, q_fg, s_all, q_all,
         jnp.full((cb, 1, 1), cnt, x3.dtype),
         jnp.zeros((cb, 1, 3), x3.dtype)], axis=2)

    @pl.when(jnp.logical_and(b == 0, i == 0))
    def _():
        o_ref[...] = jnp.zeros_like(o_ref)

    o_ref[pl.ds(i * cb, cb)] += part


def _apply_body(x_ref, m_ref, st_ref, gb_ref, o_ref, *, cb, h, w, n):
    st = st_ref[...]
    s_fg = st[:, :, 0:1]
    q_fg = st[:, :, 1:2]
    s_all = st[:, :, 2:3]
    q_all = st[:, :, 3:4]
    cnt = st[:, :, 4:5]

    sr_fg = jnp.where(cnt == 0.0, 1.0, cnt)
    cnt_bg = n - cnt
    sr_bg = jnp.where(cnt_bg == 0.0, 1.0, cnt_bg)
    s_bg = s_all - s_fg
    q_bg = q_all - q_fg
    mu_fg = s_fg / sr_fg
    mu_bg = s_bg / sr_bg
    var_fg = (q_fg - sr_fg * mu_fg * mu_fg) / n
    var_bg = (q_bg - sr_bg * mu_bg * mu_bg) / n
    a_fg = jax.lax.rsqrt(var_fg + EPS) * jnp.sqrt(sr_fg / n)
    a_bg = jax.lax.rsqrt(var_bg + EPS) * jnp.sqrt(sr_bg / n)

    gb = gb_ref[0]
    A_fg = a_fg * (1.0 + gb[:, :, 0:1])
    A_bg = a_bg * (1.0 + gb[:, :, 2:3])
    beta = gb[:, :, 1:2] + gb[:, :, 3:4]
    D_fg = beta - mu_fg * A_fg
    D_bg = beta - mu_bg * A_bg

    x3 = x_ref[0, 0].reshape(cb, h, w)
    fg = m_ref[0, 0][None] != 0.0
    a_sel = jnp.where(fg, A_fg, A_bg)
    d_sel = jnp.where(fg, D_fg, D_bg)
    res = x3 * a_sel + d_sel
    o_ref[0, 0] = res.reshape(cb * h, w)


def kernel(x, mask, fg_gamma, fg_beta, bg_gamma, bg_beta):
    B, C, H, W = x.shape
    n = float(B * H * W)
    nc = C // CB
    x4 = x.reshape(B, nc, CB * H, W)
    gb = jnp.stack([fg_gamma, fg_beta, bg_gamma, bg_beta],
                   axis=1).reshape(nc, CB, 1, 4)

    grid = (B, nc)

    stats = pl.pallas_call(
        functools.partial(_stats_body, cb=CB, h=H, w=W),
        grid=grid,
        in_specs=[
            pl.BlockSpec((1, 1, CB * H, W), lambda b, i: (b, i, 0, 0)),
            pl.BlockSpec((1, 1, H, W), lambda b, i: (b, 0, 0, 0)),
        ],
        out_specs=pl.BlockSpec((C, 1, 8), lambda b, i: (0, 0, 0)),
        out_shape=jax.ShapeDtypeStruct((C, 1, 8), jnp.float32),
        compiler_params=pltpu.CompilerParams(
            dimension_semantics=("parallel", "arbitrary"),
            vmem_limit_bytes=50 * 1024 * 1024,
        ),
    )(x4, mask)

    out = pl.pallas_call(
        functools.partial(_apply_body, cb=CB, h=H, w=W, n=n),
        grid=grid,
        in_specs=[
            pl.BlockSpec((1, 1, CB * H, W), lambda b, i: (b, i, 0, 0)),
            pl.BlockSpec((1, 1, H, W), lambda b, i: (b, 0, 0, 0)),
            pl.BlockSpec((CB, 1, 8), lambda b, i: (i, 0, 0)),
            pl.BlockSpec((1, CB, 1, 4), lambda b, i: (i, 0, 0, 0)),
        ],
        out_specs=pl.BlockSpec((1, 1, CB * H, W), lambda b, i: (b, i, 0, 0)),
        out_shape=jax.ShapeDtypeStruct((B, nc, CB * H, W), jnp.float32),
        compiler_params=pltpu.CompilerParams(
            dimension_semantics=("parallel", "arbitrary"),
            vmem_limit_bytes=50 * 1024 * 1024,
        ),
    )(x4, mask, stats, gb)

    return out.reshape(B, C, H, W)
